# baseline (speedup 1.0000x reference)
import functools

import jax
import jax.numpy as jnp
from jax import lax
from jax.experimental import pallas as pl
from jax.experimental.pallas import tpu as pltpu
from jax.experimental.pallas import tpu_sc as plsc

N = 10000
E = 320000
HID = 128
C = 40

NC = 2
NS = 16
NW = NC * NS

CHUNK = 128
CPW0 = 96
CPW1 = 62
NCH = NS * (CPW0 + CPW1)
EP = NCH * CHUNK
NP = N + 112
RPT = NP // NS
DW = 8

_sc_mesh = plsc.VectorSubcoreMesh(
    core_axis_name="c", subcore_axis_name="s", num_cores=NC, num_subcores=NS
)


@functools.partial(
    pl.kernel,
    out_type=jax.ShapeDtypeStruct((NC, NP, DW), jnp.float32),
    mesh=_sc_mesh,
    scratch_types=[
        pltpu.VMEM((CHUNK,), jnp.int32),
        pltpu.VMEM((CHUNK, DW), jnp.float32),
        pltpu.VMEM_SHARED((NP, DW), jnp.float32),
    ],
)
def _sc_degree(dst_hbm, ones_hbm, zeros_hbm, out_hbm, didx, ones_v, acc):
    c = lax.axis_index("c")
    s = lax.axis_index("s")
    pltpu.sync_copy(zeros_hbm, acc.at[pl.ds(s * RPT, RPT)])
    pltpu.sync_copy(ones_hbm, ones_v)
    plsc.subcore_barrier()
    base = jnp.where(c == 0, s * CPW0, NS * CPW0 + s * CPW1)
    nb = jnp.where(c == 0, CPW0, CPW1)

    def body(j, carry):
        pltpu.sync_copy(dst_hbm.at[base + j], didx)
        pltpu.sync_copy(ones_v, acc.at[didx], add=True)
        return carry

    lax.fori_loop(0, nb, body, 0)
    plsc.subcore_barrier()
    rows = pl.ds(s * RPT, RPT)
    pltpu.sync_copy(acc.at[rows], out_hbm.at[c, rows])


@functools.partial(
    pl.kernel,
    out_type=jax.ShapeDtypeStruct((NC, NP, HID), jnp.float32),
    mesh=_sc_mesh,
    scratch_types=[
        pltpu.VMEM((CHUNK,), jnp.int32),
        pltpu.VMEM((CHUNK,), jnp.int32),
        pltpu.VMEM((CHUNK, HID), jnp.float32),
        pltpu.VMEM_SHARED((NP, HID), jnp.float32),
        pltpu.SemaphoreType.DMA,
    ],
)
def _sc_gather_scatter(zs_hbm, src_hbm, dst_hbm, zeros_hbm, out_hbm,
                       sidx, didx, rows_v, acc, sem_g):
    c = lax.axis_index("c")
    s = lax.axis_index("s")
    pltpu.sync_copy(zeros_hbm, acc.at[pl.ds(s * RPT, RPT)])
    plsc.subcore_barrier()
    base = jnp.where(c == 0, s * CPW0, NS * CPW0 + s * CPW1)
    nb = jnp.where(c == 0, CPW0, CPW1)

    def body(j, carry):
        pltpu.sync_copy(src_hbm.at[base + j], sidx)
        pltpu.sync_copy(dst_hbm.at[base + j], didx)
        pltpu.async_copy(zs_hbm.at[sidx], rows_v, sem_g).wait()
        pltpu.sync_copy(rows_v, acc.at[didx], add=True)
        return carry

    lax.fori_loop(0, nb, body, 0)
    plsc.subcore_barrier()
    rows = pl.ds(s * RPT, RPT)
    pltpu.sync_copy(acc.at[rows], out_hbm.at[c, rows])


RB = 1000
GRID = N // RB


def _rows_spec(d=HID):
    return pl.BlockSpec((RB, d), lambda i: (i, 0))


def _full_spec(shape):
    return pl.BlockSpec(shape, lambda i: (0,) * len(shape))


def _deg_spec():
    return pl.BlockSpec((NC, RB, DW), lambda i: (0, i, 0))


def _layer_norm(x, g, b):
    mu = jnp.mean(x, axis=-1, keepdims=True)
    var = jnp.mean((x - mu) ** 2, axis=-1, keepdims=True)
    return (x - mu) * lax.rsqrt(var + 1e-5) * g + b


def _dinv_of(deg_ref):
    deg = deg_ref[0, :, 0] + deg_ref[1, :, 0] + 2.0
    return lax.rsqrt(deg)[:, None]


def _emb_block(x, w, b):
    return jnp.dot(x, w, preferred_element_type=jnp.float32) + b


def _gcn_post_block(h, u, zs, acc0, acc1, dinv, wout, bout, bgcn):
    gcn = dinv * (acc0 + acc1 + 2.0 * zs) + bgcn
    gated = jnp.tanh(gcn) * u
    return h + jnp.dot(gated, wout, preferred_element_type=jnp.float32) + bout


def _pre_block(h, ng, nb, win, bin_, sg, sb, wgcn, dinv):
    t = _layer_norm(h, ng, nb)
    a = jnp.dot(t, win, preferred_element_type=jnp.float32) + bin_
    u = 0.5 * a * (1.0 + lax.erf(a * 0.7071067811865476))
    g = _layer_norm(u, sg, sb)
    z = jnp.dot(g, wgcn, preferred_element_type=jnp.float32)
    return u, z * dinv


def _stage1_body(x_ref, deg_ref, wemb_ref, bemb_ref, ng_ref, nb_ref,
                 win_ref, bin_ref, sg_ref, sb_ref, wgcn_ref,
                 h_ref, u_ref, zs_ref):
    dinv = _dinv_of(deg_ref)
    h = _emb_block(x_ref[...], wemb_ref[...], bemb_ref[...])
    u, zs = _pre_block(h, ng_ref[...], nb_ref[...], win_ref[...], bin_ref[...],
                       sg_ref[...], sb_ref[...], wgcn_ref[...], dinv)
    h_ref[...] = h
    u_ref[...] = u
    zs_ref[...] = zs


def _stage2_body(h_ref, u_ref, zs_ref, acc_ref, deg_ref,
                 wout_ref, bout_ref, bgcn_ref,
                 ng_ref, nb_ref, win_ref, bin_ref, sg_ref, sb_ref, wgcn_ref,
                 h2_ref, u2_ref, zs2_ref):
    dinv = _dinv_of(deg_ref)
    h2 = _gcn_post_block(h_ref[...], u_ref[...], zs_ref[...],
                         acc_ref[0], acc_ref[1], dinv,
                         wout_ref[...], bout_ref[...], bgcn_ref[...])
    u2, zs2 = _pre_block(h2, ng_ref[...], nb_ref[...], win_ref[...],
                         bin_ref[...], sg_ref[...], sb_ref[...],
                         wgcn_ref[...], dinv)
    h2_ref[...] = h2
    u2_ref[...] = u2
    zs2_ref[...] = zs2


def _stage3_body(h_ref, u_ref, zs_ref, acc_ref, deg_ref,
                 wout_ref, bout_ref, bgcn_ref, wlin_ref, blin_ref, o_ref):
    dinv = _dinv_of(deg_ref)
    h2 = _gcn_post_block(h_ref[...], u_ref[...], zs_ref[...],
                         acc_ref[0], acc_ref[1], dinv,
                         wout_ref[...], bout_ref[...], bgcn_ref[...])
    o_ref[...] = (
        jnp.dot(h2, wlin_ref[...], preferred_element_type=jnp.float32)
        + blin_ref[...]
    )


_acc_spec = pl.BlockSpec((NC, RB, HID), lambda i: (0, i, 0))
_w_spec = _full_spec((HID, HID))
_b_spec = _full_spec((1, HID))
_nhid = jax.ShapeDtypeStruct((N, HID), jnp.float32)

_stage1 = pl.pallas_call(
    _stage1_body,
    grid=(GRID,),
    in_specs=[_rows_spec(), _deg_spec(), _w_spec, _b_spec,
              _b_spec, _b_spec, _w_spec, _b_spec, _b_spec, _b_spec, _w_spec],
    out_specs=[_rows_spec(), _rows_spec(), _rows_spec()],
    out_shape=[_nhid, _nhid, _nhid],
)

_stage2 = pl.pallas_call(
    _stage2_body,
    grid=(GRID,),
    in_specs=[_rows_spec(), _rows_spec(), _rows_spec(), _acc_spec, _deg_spec(),
              _w_spec, _b_spec, _b_spec,
              _b_spec, _b_spec, _w_spec, _b_spec, _b_spec, _b_spec, _w_spec],
    out_specs=[_rows_spec(), _rows_spec(), _rows_spec()],
    out_shape=[_nhid, _nhid, _nhid],
)

_stage3 = pl.pallas_call(
    _stage3_body,
    grid=(GRID,),
    in_specs=[_rows_spec(), _rows_spec(), _rows_spec(), _acc_spec, _deg_spec(),
              _w_spec, _b_spec, _b_spec, _full_spec((HID, C)), _full_spec((1, C))],
    out_specs=_rows_spec(C),
    out_shape=jax.ShapeDtypeStruct((N, C), jnp.float32),
)


def kernel(x, params, edge_index):
    f32 = jnp.float32
    src = edge_index[0].astype(jnp.int32)
    dst = edge_index[1].astype(jnp.int32)
    pad = EP - E
    srcp = jnp.concatenate([src, jnp.zeros((pad,), jnp.int32)]).reshape(
        EP // CHUNK, CHUNK)
    dstp = jnp.concatenate([dst, jnp.full((pad,), N, jnp.int32)]).reshape(
        EP // CHUNK, CHUNK)

    ones_dw = jnp.ones((CHUNK, DW), f32)
    zeros_dw = jnp.zeros((RPT, DW), f32)
    zeros_h = jnp.zeros((RPT, HID), f32)

    degp = _sc_degree(dstp, ones_dw, zeros_dw)[:, :N, :]

    p = params
    l0, l1 = p['layers']
    h0, u0, zs0 = _stage1(
        x, degp, p['Wemb'].T, p['bemb'][None, :],
        l0['norm_g'][None, :], l0['norm_b'][None, :],
        l0['Win'].T, l0['bin'][None, :],
        l0['sgu_norm_g'][None, :], l0['sgu_norm_b'][None, :], l0['Wgcn'].T,
    )
    acc0 = _sc_gather_scatter(zs0, srcp, dstp, zeros_h)[:, :N, :]
    h1, u1, zs1 = _stage2(
        h0, u0, zs0, acc0, degp,
        l0['Wout'].T, l0['bout'][None, :], l0['bgcn'][None, :],
        l1['norm_g'][None, :], l1['norm_b'][None, :],
        l1['Win'].T, l1['bin'][None, :],
        l1['sgu_norm_g'][None, :], l1['sgu_norm_b'][None, :], l1['Wgcn'].T,
    )
    acc1 = _sc_gather_scatter(zs1, srcp, dstp, zeros_h)[:, :N, :]
    return _stage3(
        h1, u1, zs1, acc1, degp,
        l1['Wout'].T, l1['bout'][None, :], l1['bgcn'][None, :],
        p['Wlin'].T, p['blin'][None, :],
    )


# speedup vs baseline: 1.0794x; 1.0794x over previous
import functools

import jax
import jax.numpy as jnp
from jax import lax
from jax.experimental import pallas as pl
from jax.experimental.pallas import tpu as pltpu
from jax.experimental.pallas import tpu_sc as plsc

N = 10000
E = 320000
HID = 128
C = 40

NC = 2
NS = 16
NW = NC * NS

CHUNK = 128
CPW0 = 96
CPW1 = 62
NCH = NS * (CPW0 + CPW1)
EP = NCH * CHUNK
NP = N + 112
RPT = NP // NS
DW = 8

_sc_mesh = plsc.VectorSubcoreMesh(
    core_axis_name="c", subcore_axis_name="s", num_cores=NC, num_subcores=NS
)


@functools.partial(
    pl.kernel,
    out_type=jax.ShapeDtypeStruct((NC, NP, DW), jnp.float32),
    mesh=_sc_mesh,
    scratch_types=[
        pltpu.VMEM((CHUNK,), jnp.int32),
        pltpu.VMEM((CHUNK, DW), jnp.float32),
        pltpu.VMEM_SHARED((NP, DW), jnp.float32),
    ],
)
def _sc_degree(dst_hbm, ones_hbm, zeros_hbm, out_hbm, didx, ones_v, acc):
    c = lax.axis_index("c")
    s = lax.axis_index("s")
    pltpu.sync_copy(zeros_hbm, acc.at[pl.ds(s * RPT, RPT)])
    pltpu.sync_copy(ones_hbm, ones_v)
    plsc.subcore_barrier()
    base = jnp.where(c == 0, s * CPW0, NS * CPW0 + s * CPW1)
    nb = jnp.where(c == 0, CPW0, CPW1)

    def body(j, carry):
        pltpu.sync_copy(dst_hbm.at[base + j], didx)
        pltpu.sync_copy(ones_v, acc.at[didx], add=True)
        return carry

    lax.fori_loop(0, nb, body, 0)
    plsc.subcore_barrier()
    rows = pl.ds(s * RPT, RPT)
    pltpu.sync_copy(acc.at[rows], out_hbm.at[c, rows])


@functools.partial(
    pl.kernel,
    out_type=jax.ShapeDtypeStruct((NC, NP, HID), jnp.float32),
    mesh=_sc_mesh,
    scratch_types=[
        pltpu.VMEM((CHUNK,), jnp.int32),
        pltpu.VMEM((CHUNK,), jnp.int32),
        pltpu.VMEM((CHUNK, HID), jnp.float32),
        pltpu.VMEM_SHARED((NP, HID), jnp.float32),
        pltpu.SemaphoreType.DMA,
    ],
)
def _sc_gather_scatter(zs_hbm, src_hbm, dst_hbm, zeros_hbm, out_hbm,
                       sidx, didx, rows_v, acc, sem_g):
    c = lax.axis_index("c")
    s = lax.axis_index("s")
    pltpu.sync_copy(zeros_hbm, acc.at[pl.ds(s * RPT, RPT)])
    plsc.subcore_barrier()
    base = jnp.where(c == 0, s * CPW0, NS * CPW0 + s * CPW1)
    nb = jnp.where(c == 0, CPW0, CPW1)

    def body(j, carry):
        pltpu.sync_copy(src_hbm.at[base + j], sidx)
        pltpu.sync_copy(dst_hbm.at[base + j], didx)
        pltpu.async_copy(zs_hbm.at[sidx], rows_v, sem_g).wait()
        pltpu.sync_copy(rows_v, acc.at[didx], add=True)
        return carry

    lax.fori_loop(0, nb, body, 0)
    plsc.subcore_barrier()
    rows = pl.ds(s * RPT, RPT)
    pltpu.sync_copy(acc.at[rows], out_hbm.at[c, rows])


RB = 1000
GRID = N // RB


def _rows_spec(d=HID):
    return pl.BlockSpec((RB, d), lambda i: (i, 0))


def _full_spec(shape):
    return pl.BlockSpec(shape, lambda i: (0,) * len(shape))


def _deg_spec():
    return pl.BlockSpec((NC, RB, DW), lambda i: (0, i, 0))


def _layer_norm(x, g, b):
    mu = jnp.mean(x, axis=-1, keepdims=True)
    var = jnp.mean((x - mu) ** 2, axis=-1, keepdims=True)
    return (x - mu) * lax.rsqrt(var + 1e-5) * g + b


def _dinv_of(deg_ref):
    deg = deg_ref[0, :, 0] + deg_ref[1, :, 0] + 2.0
    return lax.rsqrt(deg)[:, None]


def _emb_block(x, w, b):
    return jnp.dot(x, w, preferred_element_type=jnp.float32) + b


def _gcn_post_block(h, u, zs, acc0, acc1, dinv, wout, bout, bgcn):
    gcn = dinv * (acc0 + acc1 + 2.0 * zs) + bgcn
    gated = jnp.tanh(gcn) * u
    return h + jnp.dot(gated, wout, preferred_element_type=jnp.float32) + bout


def _pre_block(h, ng, nb, win, bin_, sg, sb, wgcn, dinv):
    t = _layer_norm(h, ng, nb)
    a = jnp.dot(t, win, preferred_element_type=jnp.float32) + bin_
    u = 0.5 * a * (1.0 + lax.erf(a * 0.7071067811865476))
    g = _layer_norm(u, sg, sb)
    z = jnp.dot(g, wgcn, preferred_element_type=jnp.float32)
    return u, (z if dinv is None else z * dinv)


def _stage1_body(x_ref, wemb_ref, bemb_ref, ng_ref, nb_ref,
                 win_ref, bin_ref, sg_ref, sb_ref, wgcn_ref,
                 h_ref, u_ref, z_ref):
    h = _emb_block(x_ref[...], wemb_ref[...], bemb_ref[...])
    u, z = _pre_block(h, ng_ref[...], nb_ref[...], win_ref[...], bin_ref[...],
                      sg_ref[...], sb_ref[...], wgcn_ref[...], None)
    h_ref[...] = h
    u_ref[...] = u
    z_ref[...] = z


def _scale_body(z_ref, deg_ref, zs_ref):
    zs_ref[...] = z_ref[...] * _dinv_of(deg_ref)


def _stage2_body(h_ref, u_ref, zs_ref, acc_ref, deg_ref,
                 wout_ref, bout_ref, bgcn_ref,
                 ng_ref, nb_ref, win_ref, bin_ref, sg_ref, sb_ref, wgcn_ref,
                 h2_ref, u2_ref, zs2_ref):
    dinv = _dinv_of(deg_ref)
    h2 = _gcn_post_block(h_ref[...], u_ref[...], zs_ref[...],
                         acc_ref[0], acc_ref[1], dinv,
                         wout_ref[...], bout_ref[...], bgcn_ref[...])
    u2, zs2 = _pre_block(h2, ng_ref[...], nb_ref[...], win_ref[...],
                         bin_ref[...], sg_ref[...], sb_ref[...],
                         wgcn_ref[...], dinv)
    h2_ref[...] = h2
    u2_ref[...] = u2
    zs2_ref[...] = zs2


def _stage3_body(h_ref, u_ref, zs_ref, acc_ref, deg_ref,
                 wout_ref, bout_ref, bgcn_ref, wlin_ref, blin_ref, o_ref):
    dinv = _dinv_of(deg_ref)
    h2 = _gcn_post_block(h_ref[...], u_ref[...], zs_ref[...],
                         acc_ref[0], acc_ref[1], dinv,
                         wout_ref[...], bout_ref[...], bgcn_ref[...])
    o_ref[...] = (
        jnp.dot(h2, wlin_ref[...], preferred_element_type=jnp.float32)
        + blin_ref[...]
    )


_acc_spec = pl.BlockSpec((NC, RB, HID), lambda i: (0, i, 0))
_w_spec = _full_spec((HID, HID))
_b_spec = _full_spec((1, HID))
_nhid = jax.ShapeDtypeStruct((N, HID), jnp.float32)

_stage1 = pl.pallas_call(
    _stage1_body,
    grid=(GRID,),
    in_specs=[_rows_spec(), _w_spec, _b_spec,
              _b_spec, _b_spec, _w_spec, _b_spec, _b_spec, _b_spec, _w_spec],
    out_specs=[_rows_spec(), _rows_spec(), _rows_spec()],
    out_shape=[_nhid, _nhid, _nhid],
)

_scale = pl.pallas_call(
    _scale_body,
    grid=(GRID,),
    in_specs=[_rows_spec(), _deg_spec()],
    out_specs=_rows_spec(),
    out_shape=jax.ShapeDtypeStruct((N, HID), jnp.float32),
)

_stage2 = pl.pallas_call(
    _stage2_body,
    grid=(GRID,),
    in_specs=[_rows_spec(), _rows_spec(), _rows_spec(), _acc_spec, _deg_spec(),
              _w_spec, _b_spec, _b_spec,
              _b_spec, _b_spec, _w_spec, _b_spec, _b_spec, _b_spec, _w_spec],
    out_specs=[_rows_spec(), _rows_spec(), _rows_spec()],
    out_shape=[_nhid, _nhid, _nhid],
)

_stage3 = pl.pallas_call(
    _stage3_body,
    grid=(GRID,),
    in_specs=[_rows_spec(), _rows_spec(), _rows_spec(), _acc_spec, _deg_spec(),
              _w_spec, _b_spec, _b_spec, _full_spec((HID, C)), _full_spec((1, C))],
    out_specs=_rows_spec(C),
    out_shape=jax.ShapeDtypeStruct((N, C), jnp.float32),
)


def kernel(x, params, edge_index):
    f32 = jnp.float32
    src = edge_index[0].astype(jnp.int32)
    dst = edge_index[1].astype(jnp.int32)
    pad = EP - E
    srcp = jnp.concatenate([src, jnp.zeros((pad,), jnp.int32)]).reshape(
        EP // CHUNK, CHUNK)
    dstp = jnp.concatenate([dst, jnp.full((pad,), N, jnp.int32)]).reshape(
        EP // CHUNK, CHUNK)

    ones_dw = jnp.ones((CHUNK, DW), f32)
    zeros_dw = jnp.zeros((RPT, DW), f32)
    zeros_h = jnp.zeros((RPT, HID), f32)

    degp = _sc_degree(dstp, ones_dw, zeros_dw)[:, :N, :]

    p = params
    l0, l1 = p['layers']
    h0, u0, z0 = _stage1(
        x, p['Wemb'].T, p['bemb'][None, :],
        l0['norm_g'][None, :], l0['norm_b'][None, :],
        l0['Win'].T, l0['bin'][None, :],
        l0['sgu_norm_g'][None, :], l0['sgu_norm_b'][None, :], l0['Wgcn'].T,
    )
    zs0 = _scale(z0, degp)
    acc0 = _sc_gather_scatter(zs0, srcp, dstp, zeros_h)[:, :N, :]
    h1, u1, zs1 = _stage2(
        h0, u0, zs0, acc0, degp,
        l0['Wout'].T, l0['bout'][None, :], l0['bgcn'][None, :],
        l1['norm_g'][None, :], l1['norm_b'][None, :],
        l1['Win'].T, l1['bin'][None, :],
        l1['sgu_norm_g'][None, :], l1['sgu_norm_b'][None, :], l1['Wgcn'].T,
    )
    acc1 = _sc_gather_scatter(zs1, srcp, dstp, zeros_h)[:, :N, :]
    return _stage3(
        h1, u1, zs1, acc1, degp,
        l1['Wout'].T, l1['bout'][None, :], l1['bgcn'][None, :],
        p['Wlin'].T, p['blin'][None, :],
    )
